# baseline (speedup 1.0000x reference)
import jax
import jax.numpy as jnp
from jax import lax
from jax.experimental import pallas as pl
from jax.experimental.pallas import tpu as pltpu

_K = 512
_W = 4.0
_TN = 8192


def _pass1_body(x_ref, w1_ref, b1_ref, w2_ref, b2_ref, rp_ref, bp_ref,
                cf_ref, out_ref, means_ref, seg_acc, cnt_acc):
    i = pl.program_id(0)
    nt = pl.num_programs(0)

    @pl.when(i == 0)
    def _init():
        seg_acc[...] = jnp.zeros_like(seg_acc)
        cnt_acc[...] = jnp.zeros_like(cnt_acc)

    x = x_ref[...]
    h = jnp.maximum(
        jnp.dot(x, w1_ref[...], preferred_element_type=jnp.float32)
        + b1_ref[...], 0.0)
    out = (jnp.dot(h, w2_ref[...], preferred_element_type=jnp.float32)
           + b2_ref[...])
    out_ref[...] = out

    y = (jnp.dot(out, rp_ref[...], preferred_element_type=jnp.float32)
         + bp_ref[...])
    codes = jnp.floor(y).astype(jnp.int32)
    s = jnp.sum(codes * cf_ref[...], axis=1, keepdims=True)
    bucket = jnp.bitwise_and(s, _K - 1)

    kiota = lax.broadcasted_iota(jnp.int32, (bucket.shape[0], _K), 1)
    p = (bucket == kiota).astype(jnp.bfloat16)
    hi = out.astype(jnp.bfloat16)
    lo = (out - hi.astype(jnp.float32)).astype(jnp.bfloat16)
    dn = (((0,), (0,)), ((), ()))
    seg_acc[...] += (
        lax.dot_general(p, hi, dn, preferred_element_type=jnp.float32)
        + lax.dot_general(p, lo, dn, preferred_element_type=jnp.float32))
    cnt_acc[...] += lax.dot_general(
        p, jnp.ones((p.shape[0], 8), jnp.bfloat16), dn,
        preferred_element_type=jnp.float32)

    @pl.when(i == nt - 1)
    def _fin():
        cnt = jnp.maximum(cnt_acc[:, 0:1], 1.0)
        means_ref[...] = seg_acc[...] / cnt


def _pass2_body(out_ref, means_ref, q_ref):
    out = out_ref[...]
    means = means_ref[...]
    m2 = lax.dot_general(
        jnp.ones((8, means.shape[1]), jnp.float32), means * means,
        (((1,), (1,)), ((), ())), preferred_element_type=jnp.float32,
        precision=lax.Precision.HIGHEST)
    mm = lax.dot_general(
        out, means, (((1,), (1,)), ((), ())),
        preferred_element_type=jnp.float32)
    logits = 2.0 * mm - m2[0:1, :]
    mx = jnp.max(logits, axis=1, keepdims=True)
    e = jnp.exp(logits - mx)
    q_ref[...] = e / jnp.sum(e, axis=1, keepdims=True)


def kernel(inputs, W1, b1, W2, b2, R, b_lsh, coeffs):
    n, latent = inputs.shape
    hidden = W1.shape[1]
    out_dim = W2.shape[1]
    nh = R.shape[1]
    nt = n // _TN

    rp = jnp.zeros((out_dim, 8), jnp.float32).at[:, :nh].set(R / _W)
    bp = jnp.zeros((1, 8), jnp.float32).at[0, :nh].set(b_lsh / _W)
    cf = jnp.zeros((1, 8), jnp.int32).at[0, :nh].set(coeffs)
    b1r = b1.reshape(1, hidden)
    b2r = b2.reshape(1, out_dim)

    out, means = pl.pallas_call(
        _pass1_body,
        grid=(nt,),
        in_specs=[
            pl.BlockSpec((_TN, latent), lambda i: (i, 0)),
            pl.BlockSpec((latent, hidden), lambda i: (0, 0)),
            pl.BlockSpec((1, hidden), lambda i: (0, 0)),
            pl.BlockSpec((hidden, out_dim), lambda i: (0, 0)),
            pl.BlockSpec((1, out_dim), lambda i: (0, 0)),
            pl.BlockSpec((out_dim, 8), lambda i: (0, 0)),
            pl.BlockSpec((1, 8), lambda i: (0, 0)),
            pl.BlockSpec((1, 8), lambda i: (0, 0)),
        ],
        out_specs=[
            pl.BlockSpec((_TN, out_dim), lambda i: (i, 0)),
            pl.BlockSpec((_K, out_dim), lambda i: (0, 0)),
        ],
        out_shape=[
            jax.ShapeDtypeStruct((n, out_dim), jnp.float32),
            jax.ShapeDtypeStruct((_K, out_dim), jnp.float32),
        ],
        scratch_shapes=[
            pltpu.VMEM((_K, out_dim), jnp.float32),
            pltpu.VMEM((_K, 8), jnp.float32),
        ],
    )(inputs, W1, b1r, W2, b2r, rp, bp, cf)

    q_s = pl.pallas_call(
        _pass2_body,
        grid=(nt,),
        in_specs=[
            pl.BlockSpec((_TN, out_dim), lambda i: (i, 0)),
            pl.BlockSpec((_K, out_dim), lambda i: (0, 0)),
        ],
        out_specs=pl.BlockSpec((_TN, _K), lambda i: (i, 0)),
        out_shape=jax.ShapeDtypeStruct((n, _K), jnp.float32),
    )(out, means)

    return (q_s, means)


# speedup vs baseline: 1.1037x; 1.1037x over previous
import jax
import jax.numpy as jnp
from jax import lax
from jax.experimental import pallas as pl
from jax.experimental.pallas import tpu as pltpu

_K = 512
_W = 4.0
_TN = 2048


def _body(x_ref, w1_ref, b1_ref, w2_ref, b2_ref, rp_ref, bp_ref,
          cf_ref, q_ref, means_ref, out_buf, seg_acc, cnt_acc, m2_buf):
    i = pl.program_id(0)
    nt = pl.num_programs(0) // 2
    tn = x_ref.shape[0]

    @pl.when(i == 0)
    def _init():
        seg_acc[...] = jnp.zeros_like(seg_acc)
        cnt_acc[...] = jnp.zeros_like(cnt_acc)

    @pl.when(i < nt)
    def _phase0():
        x = x_ref[...]
        h = jnp.maximum(
            jnp.dot(x, w1_ref[...], preferred_element_type=jnp.float32)
            + b1_ref[...], 0.0)
        out = (jnp.dot(h, w2_ref[...], preferred_element_type=jnp.float32)
               + b2_ref[...])
        out_buf[pl.ds(i * tn, tn), :] = out

        y = (jnp.dot(out, rp_ref[...], preferred_element_type=jnp.float32)
             + bp_ref[...])
        codes = jnp.floor(y).astype(jnp.int32)
        s = jnp.sum(codes * cf_ref[...], axis=1, keepdims=True)
        bucket = jnp.bitwise_and(s, _K - 1)

        kiota = lax.broadcasted_iota(jnp.int32, (tn, _K), 1)
        p = (bucket == kiota).astype(jnp.bfloat16)
        hi = out.astype(jnp.bfloat16)
        lo = (out - hi.astype(jnp.float32)).astype(jnp.bfloat16)
        dn = (((0,), (0,)), ((), ()))
        seg_acc[...] += (
            lax.dot_general(p, hi, dn, preferred_element_type=jnp.float32)
            + lax.dot_general(p, lo, dn, preferred_element_type=jnp.float32))
        cnt_acc[...] += lax.dot_general(
            p, jnp.ones((tn, 8), jnp.bfloat16), dn,
            preferred_element_type=jnp.float32)

        @pl.when(i == nt - 1)
        def _fin():
            cnt = jnp.maximum(cnt_acc[:, 0:1], 1.0)
            means = seg_acc[...] / cnt
            means_ref[...] = means
            m2_buf[...] = lax.dot_general(
                jnp.ones((8, means.shape[1]), jnp.float32), means * means,
                (((1,), (1,)), ((), ())), preferred_element_type=jnp.float32,
                precision=lax.Precision.HIGHEST)

    @pl.when(i >= nt)
    def _phase1():
        j = i - nt
        out = out_buf[pl.ds(j * tn, tn), :]
        means = means_ref[...]
        mm = lax.dot_general(
            out, means, (((1,), (1,)), ((), ())),
            preferred_element_type=jnp.float32)
        logits = 2.0 * mm - m2_buf[0:1, :]
        mx = jnp.max(logits, axis=1, keepdims=True)
        e = jnp.exp(logits - mx)
        q_ref[...] = e / jnp.sum(e, axis=1, keepdims=True)


def kernel(inputs, W1, b1, W2, b2, R, b_lsh, coeffs):
    n, latent = inputs.shape
    hidden = W1.shape[1]
    out_dim = W2.shape[1]
    nh = R.shape[1]
    nt = n // _TN

    rp = jnp.zeros((out_dim, 8), jnp.float32).at[:, :nh].set(R / _W)
    bp = jnp.zeros((1, 8), jnp.float32).at[0, :nh].set(b_lsh / _W)
    cf = jnp.zeros((1, 8), jnp.int32).at[0, :nh].set(coeffs)
    b1r = b1.reshape(1, hidden)
    b2r = b2.reshape(1, out_dim)

    q_s, means = pl.pallas_call(
        _body,
        grid=(2 * nt,),
        in_specs=[
            pl.BlockSpec((_TN, latent), lambda i: (jnp.minimum(i, nt - 1), 0)),
            pl.BlockSpec((latent, hidden), lambda i: (0, 0)),
            pl.BlockSpec((1, hidden), lambda i: (0, 0)),
            pl.BlockSpec((hidden, out_dim), lambda i: (0, 0)),
            pl.BlockSpec((1, out_dim), lambda i: (0, 0)),
            pl.BlockSpec((out_dim, 8), lambda i: (0, 0)),
            pl.BlockSpec((1, 8), lambda i: (0, 0)),
            pl.BlockSpec((1, 8), lambda i: (0, 0)),
        ],
        out_specs=[
            pl.BlockSpec((_TN, _K), lambda i: (jnp.maximum(i - nt, 0), 0)),
            pl.BlockSpec((_K, out_dim), lambda i: (0, 0)),
        ],
        out_shape=[
            jax.ShapeDtypeStruct((n, _K), jnp.float32),
            jax.ShapeDtypeStruct((_K, out_dim), jnp.float32),
        ],
        scratch_shapes=[
            pltpu.VMEM((n, out_dim), jnp.float32),
            pltpu.VMEM((_K, out_dim), jnp.float32),
            pltpu.VMEM((_K, 8), jnp.float32),
            pltpu.VMEM((8, _K), jnp.float32),
        ],
    )(inputs, W1, b1r, W2, b2r, rp, bp, cf)

    return (q_s, means)


# speedup vs baseline: 1.1382x; 1.0313x over previous
import jax
import jax.numpy as jnp
from jax import lax
from jax.experimental import pallas as pl
from jax.experimental.pallas import tpu as pltpu

_K = 512
_W = 4.0
_TN = 4096


def _body(x_ref, w1_ref, b1_ref, w2_ref, b2_ref, rp_ref, bp_ref,
          cf_ref, q_ref, means_ref, out_buf, seg_acc, cnt_acc, m2_buf):
    i = pl.program_id(0)
    nt = pl.num_programs(0) // 2
    tn = x_ref.shape[0]

    @pl.when(i == 0)
    def _init():
        seg_acc[...] = jnp.zeros_like(seg_acc)
        cnt_acc[...] = jnp.zeros_like(cnt_acc)

    @pl.when(i < nt)
    def _phase0():
        x = x_ref[...]
        h = jnp.maximum(
            jnp.dot(x, w1_ref[...], preferred_element_type=jnp.float32)
            + b1_ref[...], 0.0)
        out = (jnp.dot(h, w2_ref[...], preferred_element_type=jnp.float32)
               + b2_ref[...])
        out_buf[pl.ds(i * tn, tn), :] = out

        y = (jnp.dot(out, rp_ref[...], preferred_element_type=jnp.float32)
             + bp_ref[...])
        codes = jnp.floor(y).astype(jnp.int32)
        s = jnp.sum(codes * cf_ref[...], axis=1, keepdims=True)
        bucket = jnp.bitwise_and(s, _K - 1)

        kiota = lax.broadcasted_iota(jnp.int32, (tn, _K), 1)
        p = (bucket == kiota).astype(jnp.bfloat16)
        hi = out.astype(jnp.bfloat16)
        lo = (out - hi.astype(jnp.float32)).astype(jnp.bfloat16)
        dn = (((0,), (0,)), ((), ()))
        seg_acc[...] += (
            lax.dot_general(p, hi, dn, preferred_element_type=jnp.float32)
            + lax.dot_general(p, lo, dn, preferred_element_type=jnp.float32))
        cnt_acc[...] += lax.dot_general(
            p, jnp.ones((tn, 8), jnp.bfloat16), dn,
            preferred_element_type=jnp.float32)

        @pl.when(i == nt - 1)
        def _fin():
            cnt = jnp.maximum(cnt_acc[:, 0:1], 1.0)
            means = seg_acc[...] / cnt
            means_ref[...] = means
            m2_buf[...] = lax.dot_general(
                jnp.ones((8, means.shape[1]), jnp.float32), means * means,
                (((1,), (1,)), ((), ())), preferred_element_type=jnp.float32,
                precision=lax.Precision.HIGHEST)

    @pl.when(i >= nt)
    def _phase1():
        j = i - nt
        out = out_buf[pl.ds(j * tn, tn), :]
        means = means_ref[...]
        mm = lax.dot_general(
            out, means, (((1,), (1,)), ((), ())),
            preferred_element_type=jnp.float32)
        logits = 2.0 * mm - m2_buf[0:1, :]
        mx = jnp.max(logits, axis=1, keepdims=True)
        e = jnp.exp(logits - mx)
        q_ref[...] = e / jnp.sum(e, axis=1, keepdims=True)


def kernel(inputs, W1, b1, W2, b2, R, b_lsh, coeffs):
    n, latent = inputs.shape
    hidden = W1.shape[1]
    out_dim = W2.shape[1]
    nh = R.shape[1]
    nt = n // _TN

    rp = jnp.zeros((out_dim, 8), jnp.float32).at[:, :nh].set(R / _W)
    bp = jnp.zeros((1, 8), jnp.float32).at[0, :nh].set(b_lsh / _W)
    cf = jnp.zeros((1, 8), jnp.int32).at[0, :nh].set(coeffs)
    b1r = b1.reshape(1, hidden)
    b2r = b2.reshape(1, out_dim)

    q_s, means = pl.pallas_call(
        _body,
        grid=(2 * nt,),
        in_specs=[
            pl.BlockSpec((_TN, latent), lambda i: (jnp.minimum(i, nt - 1), 0)),
            pl.BlockSpec((latent, hidden), lambda i: (0, 0)),
            pl.BlockSpec((1, hidden), lambda i: (0, 0)),
            pl.BlockSpec((hidden, out_dim), lambda i: (0, 0)),
            pl.BlockSpec((1, out_dim), lambda i: (0, 0)),
            pl.BlockSpec((out_dim, 8), lambda i: (0, 0)),
            pl.BlockSpec((1, 8), lambda i: (0, 0)),
            pl.BlockSpec((1, 8), lambda i: (0, 0)),
        ],
        out_specs=[
            pl.BlockSpec((_TN, _K), lambda i: (jnp.maximum(i - nt, 0), 0)),
            pl.BlockSpec((_K, out_dim), lambda i: (0, 0)),
        ],
        out_shape=[
            jax.ShapeDtypeStruct((n, _K), jnp.float32),
            jax.ShapeDtypeStruct((_K, out_dim), jnp.float32),
        ],
        scratch_shapes=[
            pltpu.VMEM((n, out_dim), jnp.float32),
            pltpu.VMEM((_K, out_dim), jnp.float32),
            pltpu.VMEM((_K, 8), jnp.float32),
            pltpu.VMEM((8, _K), jnp.float32),
        ],
    )(inputs, W1, b1r, W2, b2r, rp, bp, cf)

    return (q_s, means)
